# baseline (speedup 1.0000x reference)
import functools

import jax
import jax.numpy as jnp
from jax.experimental import pallas as pl

NSLAB = 4


def _sub_slab(r, slab, pref, opref, sufr, osufr, ctx_v, ctxo_v, out, n_cls, n_ctx):
    @pl.when(slab == 0)
    def _():
        out[r, 0:n_cls, :] = pref[:, 0, :]
        out[r, n_cls:, :] = opref[:, 0, :]

    @pl.when(jnp.logical_and(slab >= 1, slab < 1 + n_ctx))
    def _():
        j = jnp.clip(slab - 1, 0, n_ctx - 1)
        out[r, 0:n_cls, :] = jnp.broadcast_to(ctx_v[pl.ds(j, 1), :], (n_cls, ctx_v.shape[1]))
        out[r, n_cls:, :] = jnp.broadcast_to(ctxo_v[pl.ds(j, 1), :], (out.shape[1] - n_cls, ctxo_v.shape[1]))

    @pl.when(slab >= 1 + n_ctx)
    def _():
        out[r, 0:n_cls, :] = sufr[0]
        out[r, n_cls:, :] = osufr[0]


def _body(pref, opref, *rest, n_cls, n_ctx):
    sufs = rest[0:NSLAB]
    osufs = rest[NSLAB:2 * NSLAB]
    ctx_v, ctxo_v, out = rest[2 * NSLAB], rest[2 * NSLAB + 1], rest[2 * NSLAB + 2]
    b = pl.program_id(0)
    for r in range(NSLAB):
        _sub_slab(r, NSLAB * b + r, pref, opref, sufs[r], osufs[r],
                  ctx_v, ctxo_v, out, n_cls, n_ctx)


def kernel(ctx, ctx_ood, token_prefix, token_suffix, ood_token_prefix, ood_token_suffix):
    n_cls = token_prefix.shape[0]
    n_ex = ood_token_prefix.shape[0]
    n_ctx, ctx_dim = ctx.shape
    suf_len = token_suffix.shape[1]
    seq = 1 + n_ctx + suf_len
    s0 = 1 + n_ctx

    sufT = token_suffix.transpose(1, 0, 2)
    osufT = ood_token_suffix.transpose(1, 0, 2)

    def suf_idx(r):
        return lambda b: (jnp.clip(NSLAB * b + r - s0, 0, suf_len - 1), 0, 0)

    zero_idx = lambda b: (0, 0, 0)

    outT = pl.pallas_call(
        functools.partial(_body, n_cls=n_cls, n_ctx=n_ctx),
        grid=((seq + NSLAB - 1) // NSLAB,),
        in_specs=[
            pl.BlockSpec((n_cls, 1, ctx_dim), zero_idx),
            pl.BlockSpec((n_ex, 1, ctx_dim), zero_idx),
        ] + [
            pl.BlockSpec((1, n_cls, ctx_dim), suf_idx(r)) for r in range(NSLAB)
        ] + [
            pl.BlockSpec((1, n_ex, ctx_dim), suf_idx(r)) for r in range(NSLAB)
        ] + [
            pl.BlockSpec((n_ctx, ctx_dim), lambda b: (0, 0)),
            pl.BlockSpec((n_ctx, ctx_dim), lambda b: (0, 0)),
        ],
        out_specs=pl.BlockSpec((NSLAB, n_cls + n_ex, ctx_dim), lambda b: (b, 0, 0)),
        out_shape=jax.ShapeDtypeStruct((seq, n_cls + n_ex, ctx_dim), ctx.dtype),
    )(token_prefix, ood_token_prefix,
      *([sufT] * NSLAB), *([osufT] * NSLAB), ctx, ctx_ood)
    return outT.transpose(1, 0, 2)


# speedup vs baseline: 1.0323x; 1.0323x over previous
import functools

import jax
import jax.numpy as jnp
from jax.experimental import pallas as pl

NSLAB = 3


def _sub_slab(r, slab, pref, opref, sufr, osufr, ctx_v, ctxo_v, out, n_cls, n_ctx):
    @pl.when(slab == 0)
    def _():
        out[r, 0:n_cls, :] = pref[:, 0, :]
        out[r, n_cls:, :] = opref[:, 0, :]

    @pl.when(jnp.logical_and(slab >= 1, slab < 1 + n_ctx))
    def _():
        j = jnp.clip(slab - 1, 0, n_ctx - 1)
        out[r, 0:n_cls, :] = jnp.broadcast_to(ctx_v[pl.ds(j, 1), :], (n_cls, ctx_v.shape[1]))
        out[r, n_cls:, :] = jnp.broadcast_to(ctxo_v[pl.ds(j, 1), :], (out.shape[1] - n_cls, ctxo_v.shape[1]))

    @pl.when(slab >= 1 + n_ctx)
    def _():
        out[r, 0:n_cls, :] = sufr[0]
        out[r, n_cls:, :] = osufr[0]


def _body(pref, opref, *rest, n_cls, n_ctx):
    sufs = rest[0:NSLAB]
    osufs = rest[NSLAB:2 * NSLAB]
    ctx_v, ctxo_v, out = rest[2 * NSLAB], rest[2 * NSLAB + 1], rest[2 * NSLAB + 2]
    b = pl.program_id(0)
    for r in range(NSLAB):
        _sub_slab(r, NSLAB * b + r, pref, opref, sufs[r], osufs[r],
                  ctx_v, ctxo_v, out, n_cls, n_ctx)


def kernel(ctx, ctx_ood, token_prefix, token_suffix, ood_token_prefix, ood_token_suffix):
    n_cls = token_prefix.shape[0]
    n_ex = ood_token_prefix.shape[0]
    n_ctx, ctx_dim = ctx.shape
    suf_len = token_suffix.shape[1]
    seq = 1 + n_ctx + suf_len
    s0 = 1 + n_ctx

    sufT = token_suffix.transpose(1, 0, 2)
    osufT = ood_token_suffix.transpose(1, 0, 2)

    def suf_idx(r):
        return lambda b: (jnp.clip(NSLAB * b + r - s0, 0, suf_len - 1), 0, 0)

    zero_idx = lambda b: (0, 0, 0)

    outT = pl.pallas_call(
        functools.partial(_body, n_cls=n_cls, n_ctx=n_ctx),
        grid=((seq + NSLAB - 1) // NSLAB,),
        in_specs=[
            pl.BlockSpec((n_cls, 1, ctx_dim), zero_idx),
            pl.BlockSpec((n_ex, 1, ctx_dim), zero_idx),
        ] + [
            pl.BlockSpec((1, n_cls, ctx_dim), suf_idx(r)) for r in range(NSLAB)
        ] + [
            pl.BlockSpec((1, n_ex, ctx_dim), suf_idx(r)) for r in range(NSLAB)
        ] + [
            pl.BlockSpec((n_ctx, ctx_dim), lambda b: (0, 0)),
            pl.BlockSpec((n_ctx, ctx_dim), lambda b: (0, 0)),
        ],
        out_specs=pl.BlockSpec((NSLAB, n_cls + n_ex, ctx_dim), lambda b: (b, 0, 0)),
        out_shape=jax.ShapeDtypeStruct((seq, n_cls + n_ex, ctx_dim), ctx.dtype),
    )(token_prefix, ood_token_prefix,
      *([sufT] * NSLAB), *([osufT] * NSLAB), ctx, ctx_ood)
    return outT.transpose(1, 0, 2)


# speedup vs baseline: 1.0390x; 1.0065x over previous
import functools

import jax
import jax.numpy as jnp
from jax.experimental import pallas as pl

NSLAB = 2


def _sub_slab(r, slab, pref, opref, sufr, osufr, ctx_v, ctxo_v, out, n_cls, n_ctx):
    @pl.when(slab == 0)
    def _():
        out[r, 0:n_cls, :] = pref[:, 0, :]
        out[r, n_cls:, :] = opref[:, 0, :]

    @pl.when(jnp.logical_and(slab >= 1, slab < 1 + n_ctx))
    def _():
        j = jnp.clip(slab - 1, 0, n_ctx - 1)
        out[r, 0:n_cls, :] = jnp.broadcast_to(ctx_v[pl.ds(j, 1), :], (n_cls, ctx_v.shape[1]))
        out[r, n_cls:, :] = jnp.broadcast_to(ctxo_v[pl.ds(j, 1), :], (out.shape[1] - n_cls, ctxo_v.shape[1]))

    @pl.when(slab >= 1 + n_ctx)
    def _():
        out[r, 0:n_cls, :] = sufr[0]
        out[r, n_cls:, :] = osufr[0]


def _body(pref, opref, *rest, n_cls, n_ctx):
    sufs = rest[0:NSLAB]
    osufs = rest[NSLAB:2 * NSLAB]
    ctx_v, ctxo_v, out = rest[2 * NSLAB], rest[2 * NSLAB + 1], rest[2 * NSLAB + 2]
    b = pl.program_id(0)
    for r in range(NSLAB):
        _sub_slab(r, NSLAB * b + r, pref, opref, sufs[r], osufs[r],
                  ctx_v, ctxo_v, out, n_cls, n_ctx)


def kernel(ctx, ctx_ood, token_prefix, token_suffix, ood_token_prefix, ood_token_suffix):
    n_cls = token_prefix.shape[0]
    n_ex = ood_token_prefix.shape[0]
    n_ctx, ctx_dim = ctx.shape
    suf_len = token_suffix.shape[1]
    seq = 1 + n_ctx + suf_len
    s0 = 1 + n_ctx

    sufT = token_suffix.transpose(1, 0, 2)
    osufT = ood_token_suffix.transpose(1, 0, 2)

    def suf_idx(r):
        return lambda b: (jnp.clip(NSLAB * b + r - s0, 0, suf_len - 1), 0, 0)

    zero_idx = lambda b: (0, 0, 0)

    outT = pl.pallas_call(
        functools.partial(_body, n_cls=n_cls, n_ctx=n_ctx),
        grid=((seq + NSLAB - 1) // NSLAB,),
        in_specs=[
            pl.BlockSpec((n_cls, 1, ctx_dim), zero_idx),
            pl.BlockSpec((n_ex, 1, ctx_dim), zero_idx),
        ] + [
            pl.BlockSpec((1, n_cls, ctx_dim), suf_idx(r)) for r in range(NSLAB)
        ] + [
            pl.BlockSpec((1, n_ex, ctx_dim), suf_idx(r)) for r in range(NSLAB)
        ] + [
            pl.BlockSpec((n_ctx, ctx_dim), lambda b: (0, 0)),
            pl.BlockSpec((n_ctx, ctx_dim), lambda b: (0, 0)),
        ],
        out_specs=pl.BlockSpec((NSLAB, n_cls + n_ex, ctx_dim), lambda b: (b, 0, 0)),
        out_shape=jax.ShapeDtypeStruct((seq, n_cls + n_ex, ctx_dim), ctx.dtype),
    )(token_prefix, ood_token_prefix,
      *([sufT] * NSLAB), *([osufT] * NSLAB), ctx, ctx_ood)
    return outT.transpose(1, 0, 2)


# speedup vs baseline: 1.0433x; 1.0041x over previous
import functools

import jax
import jax.numpy as jnp
from jax.experimental import pallas as pl

NSLAB = 2


def _sub_slab(r, slab, pref, opref, sufr, osufr, ctx_v, ctxo_v, out, n_cls, n_ctx):
    @pl.when(slab == 0)
    def _():
        out[r, 0:n_cls, :] = pref[:, 0, :]
        out[r, n_cls:, :] = opref[:, 0, :]

    @pl.when(jnp.logical_and(slab >= 1, slab < 1 + n_ctx))
    def _():
        j = jnp.clip(slab - 1, 0, n_ctx - 1)
        out[r, 0:n_cls, :] = jnp.broadcast_to(ctx_v[pl.ds(j, 1), :], (n_cls, ctx_v.shape[1]))
        out[r, n_cls:, :] = jnp.broadcast_to(ctxo_v[pl.ds(j, 1), :], (out.shape[1] - n_cls, ctxo_v.shape[1]))

    @pl.when(slab >= 1 + n_ctx)
    def _():
        out[r, 0:n_cls, :] = sufr[0]
        out[r, n_cls:, :] = osufr[0]


def _body(pref, opref, *rest, n_cls, n_ctx):
    sufs = rest[0:NSLAB]
    osufs = rest[NSLAB:2 * NSLAB]
    ctx_v, ctxo_v, out = rest[2 * NSLAB], rest[2 * NSLAB + 1], rest[2 * NSLAB + 2]
    bb = pl.num_programs(0) - 1 - pl.program_id(0)
    for r in range(NSLAB):
        _sub_slab(r, NSLAB * bb + r, pref, opref, sufs[r], osufs[r],
                  ctx_v, ctxo_v, out, n_cls, n_ctx)


def kernel(ctx, ctx_ood, token_prefix, token_suffix, ood_token_prefix, ood_token_suffix):
    n_cls = token_prefix.shape[0]
    n_ex = ood_token_prefix.shape[0]
    n_ctx, ctx_dim = ctx.shape
    suf_len = token_suffix.shape[1]
    seq = 1 + n_ctx + suf_len
    s0 = 1 + n_ctx

    sufT = token_suffix.transpose(1, 0, 2)
    osufT = ood_token_suffix.transpose(1, 0, 2)

    nblk = (seq + NSLAB - 1) // NSLAB

    def suf_idx(r):
        return lambda b: (jnp.clip(NSLAB * (nblk - 1 - b) + r - s0, 0, suf_len - 1), 0, 0)

    zero_idx = lambda b: (0, 0, 0)

    outT = pl.pallas_call(
        functools.partial(_body, n_cls=n_cls, n_ctx=n_ctx),
        grid=(nblk,),
        in_specs=[
            pl.BlockSpec((n_cls, 1, ctx_dim), zero_idx),
            pl.BlockSpec((n_ex, 1, ctx_dim), zero_idx),
        ] + [
            pl.BlockSpec((1, n_cls, ctx_dim), suf_idx(r)) for r in range(NSLAB)
        ] + [
            pl.BlockSpec((1, n_ex, ctx_dim), suf_idx(r)) for r in range(NSLAB)
        ] + [
            pl.BlockSpec((n_ctx, ctx_dim), lambda b: (0, 0)),
            pl.BlockSpec((n_ctx, ctx_dim), lambda b: (0, 0)),
        ],
        out_specs=pl.BlockSpec((NSLAB, n_cls + n_ex, ctx_dim), lambda b: (nblk - 1 - b, 0, 0)),
        out_shape=jax.ShapeDtypeStruct((seq, n_cls + n_ex, ctx_dim), ctx.dtype),
    )(token_prefix, ood_token_prefix,
      *([sufT] * NSLAB), *([osufT] * NSLAB), ctx, ctx_ood)
    return outT.transpose(1, 0, 2)
